# baseline (speedup 1.0000x reference)
import functools

import jax
import jax.numpy as jnp
from jax.experimental import pallas as pl

HEADS = 4
H = 32
G = 1024


def _ln_relu(x, g, b, eps=1e-5):
    m = jnp.mean(x, axis=-1, keepdims=True)
    v = jnp.mean((x - m) ** 2, axis=-1, keepdims=True)
    return jax.nn.relu((x - m) / jnp.sqrt(v + eps) * g + b)


def _in_mlp_body(x_ref, w0_ref, b0_ref, g0_ref, be0_ref, w1_ref, b1_ref,
                 g1_ref, be1_ref, o_ref):
    x = x_ref[...]
    h = _ln_relu(jnp.dot(x, w0_ref[...], preferred_element_type=jnp.float32)
                 + b0_ref[...], g0_ref[...], be0_ref[...])
    h = _ln_relu(jnp.dot(h, w1_ref[...], preferred_element_type=jnp.float32)
                 + b1_ref[...], g1_ref[...], be1_ref[...])
    o_ref[...] = h


def _in_mlp(x, p0, p1):
    n = x.shape[0]
    blk = 2048
    grid = (n + blk - 1) // blk
    return pl.pallas_call(
        _in_mlp_body,
        grid=(grid,),
        in_specs=[
            pl.BlockSpec((blk, x.shape[1]), lambda i: (i, 0)),
            pl.BlockSpec(p0["W"].shape, lambda i: (0, 0)),
            pl.BlockSpec(p0["b"].shape, lambda i: (0,)),
            pl.BlockSpec(p0["g"].shape, lambda i: (0,)),
            pl.BlockSpec(p0["beta"].shape, lambda i: (0,)),
            pl.BlockSpec(p1["W"].shape, lambda i: (0, 0)),
            pl.BlockSpec(p1["b"].shape, lambda i: (0,)),
            pl.BlockSpec(p1["g"].shape, lambda i: (0,)),
            pl.BlockSpec(p1["beta"].shape, lambda i: (0,)),
        ],
        out_specs=pl.BlockSpec((blk, H), lambda i: (i, 0)),
        out_shape=jax.ShapeDtypeStruct((n, H), jnp.float32),
    )(x, p0["W"], p0["b"], p0["g"], p0["beta"],
      p1["W"], p1["b"], p1["g"], p1["beta"])


def _out_mlp_body(g_ref, w0_ref, b0_ref, g0_ref, be0_ref, w1_ref, b1_ref,
                  g1_ref, be1_ref, w2_ref, b2_ref, o_ref):
    h = _ln_relu(jnp.dot(g_ref[...], w0_ref[...],
                         preferred_element_type=jnp.float32) + b0_ref[...],
                 g0_ref[...], be0_ref[...])
    h = _ln_relu(jnp.dot(h, w1_ref[...], preferred_element_type=jnp.float32)
                 + b1_ref[...], g1_ref[...], be1_ref[...])
    o_ref[...] = jnp.dot(h, w2_ref[...],
                         preferred_element_type=jnp.float32) + b2_ref[...]


def _out_mlp(gfeat, p0, p1, p2):
    return pl.pallas_call(
        _out_mlp_body,
        out_shape=jax.ShapeDtypeStruct((G, 5), jnp.float32),
    )(gfeat, p0["W"], p0["b"], p0["g"], p0["beta"],
      p1["W"], p1["b"], p1["g"], p1["beta"], p2["W"], p2["b"])


def _transformer_conv(x, src, dst, p):
    n = x.shape[0]
    q = (x @ p["q"]["W"] + p["q"]["b"]).reshape(n, HEADS, H)
    k = (x @ p["k"]["W"] + p["k"]["b"]).reshape(n, HEADS, H)
    v = (x @ p["v"]["W"] + p["v"]["b"]).reshape(n, HEADS, H)
    alpha = jnp.sum(q[dst] * k[src], axis=-1) / jnp.sqrt(float(H))
    amax = jax.ops.segment_max(alpha, dst, num_segments=n)
    amax = jnp.where(jnp.isfinite(amax), amax, 0.0)
    ex = jnp.exp(alpha - amax[dst])
    denom = jax.ops.segment_sum(ex, dst, num_segments=n)
    attn = ex / (denom[dst] + 1e-16)
    msg = v[src] * attn[:, :, None]
    out = jax.ops.segment_sum(msg, dst, num_segments=n).reshape(n, HEADS * H)
    return out + x @ p["skip"]["W"] + p["skip"]["b"]


def _multi_aggr(x, batch):
    s = jax.ops.segment_sum(x, batch, num_segments=G)
    cnt = jax.ops.segment_sum(jnp.ones((x.shape[0],), x.dtype), batch,
                              num_segments=G)[:, None]
    safe = jnp.maximum(cnt, 1.0)
    mean = s / safe
    mn = jax.ops.segment_min(x, batch, num_segments=G)
    mx = jax.ops.segment_max(x, batch, num_segments=G)
    mn = jnp.where(cnt > 0, mn, 0.0)
    mx = jnp.where(cnt > 0, mx, 0.0)
    mean2 = jax.ops.segment_sum(x * x, batch, num_segments=G) / safe
    var = mean2 - mean ** 2
    std = jnp.sqrt(jnp.clip(var, 1e-5))
    return jnp.concatenate([s, mean, mn, mx, std], axis=-1)


def kernel(x, batch, edge_index, params):
    src, dst = edge_index[0], edge_index[1]
    p = params
    h = _in_mlp(x, p["in0"], p["in1"])
    h = _transformer_conv(h, src, dst, p["t0"])
    h = _transformer_conv(h, src, dst, p["t1"])
    g = _multi_aggr(h, batch)
    return _out_mlp(g, p["r0"], p["r1"], p["r2"])
